# Initial kernel scaffold; baseline (speedup 1.0000x reference)
import jax
import jax.numpy as jnp
from jax.experimental import pallas as pl


def kernel(x, A_sample, b_sample, A_patch, b_patch, ln_weight, ln_bias):
    raise NotImplementedError("write your pallas kernel here")


# speedup vs baseline: 3.4774x; 3.4774x over previous
import functools

import jax
import jax.numpy as jnp
from jax.experimental import pallas as pl

_B, _V, _S = 16, 64, 2048
_SAMPLE_DIM, _PATCH_DIM = 128, 256
_WINDOW, _STEP = 32, 16
_C = _S // _STEP
_T = (_S - _WINDOW + _STEP - 1) // _STEP


def _tok_kernel(x_ref, As_ref, bs_ref, A1_ref, A2_ref, bp_ref, lnw_ref,
                lnb_ref, o_ref):
    xb = x_ref[0]
    pre = jax.lax.dot_general(xb, As_ref[...], (((0,), (0,)), ((), ())),
                              preferred_element_type=jnp.float32)
    enc = jnp.floor(pre + bs_ref[...])
    E = enc.reshape(_C, _STEP * _SAMPLE_DIM)
    t1 = jnp.dot(E, A1_ref[...],
                 preferred_element_type=jnp.float32)
    t2 = jnp.dot(E, A2_ref[...],
                 preferred_element_type=jnp.float32)
    t2s = jnp.roll(t2, -1, axis=0)
    tok = jnp.floor(t1 + t2s + bp_ref[...])
    m = jnp.mean(tok, axis=1, keepdims=True)
    c = tok - m
    v = jnp.mean(c * c, axis=1, keepdims=True)
    o_ref[0] = c * jax.lax.rsqrt(v + 1e-5) * lnw_ref[...] + lnb_ref[...]


@functools.partial(jax.jit, static_argnames=("interpret",))
def kernel(x, A_sample, b_sample, A_patch, b_patch, ln_weight, ln_bias,
           interpret=False):
    Ap = A_patch.reshape(_SAMPLE_DIM, _WINDOW, _PATCH_DIM)
    Ap = Ap.transpose(1, 0, 2)
    A1 = Ap[:_STEP].reshape(_STEP * _SAMPLE_DIM, _PATCH_DIM)
    A2 = Ap[_STEP:].reshape(_STEP * _SAMPLE_DIM, _PATCH_DIM)
    bs = b_sample.reshape(1, _SAMPLE_DIM)
    bp = b_patch.reshape(1, _PATCH_DIM)
    lnw = ln_weight.reshape(1, _PATCH_DIM)
    lnb = ln_bias.reshape(1, _PATCH_DIM)

    out = pl.pallas_call(
        _tok_kernel,
        grid=(_B,),
        in_specs=[
            pl.BlockSpec((1, _V, _S), lambda b: (b, 0, 0)),
            pl.BlockSpec((_V, _SAMPLE_DIM), lambda b: (0, 0)),
            pl.BlockSpec((1, _SAMPLE_DIM), lambda b: (0, 0)),
            pl.BlockSpec((_STEP * _SAMPLE_DIM, _PATCH_DIM), lambda b: (0, 0)),
            pl.BlockSpec((_STEP * _SAMPLE_DIM, _PATCH_DIM), lambda b: (0, 0)),
            pl.BlockSpec((1, _PATCH_DIM), lambda b: (0, 0)),
            pl.BlockSpec((1, _PATCH_DIM), lambda b: (0, 0)),
            pl.BlockSpec((1, _PATCH_DIM), lambda b: (0, 0)),
        ],
        out_specs=pl.BlockSpec((1, _C, _PATCH_DIM), lambda b: (b, 0, 0)),
        out_shape=jax.ShapeDtypeStruct((_B, _C, _PATCH_DIM), jnp.float32),
        interpret=interpret,
    )(x, A_sample, bs, A1, A2, bp, lnw, lnb)
    return out[:, :_T, :]


# speedup vs baseline: 3.4941x; 1.0048x over previous
import functools

import jax
import jax.numpy as jnp
from jax.experimental import pallas as pl

_B, _V, _S = 16, 64, 2048
_SAMPLE_DIM, _PATCH_DIM = 128, 256
_WINDOW, _STEP = 32, 16
_C = _S // _STEP
_T = (_S - _WINDOW + _STEP - 1) // _STEP
_K2 = _STEP * _SAMPLE_DIM


def _tok_kernel(x_ref, As_ref, bs_ref, A12_ref, bp_ref, lnw_ref,
                lnb_ref, o_ref):
    xb = x_ref[0]
    pre = jax.lax.dot_general(xb, As_ref[...], (((0,), (0,)), ((), ())),
                              preferred_element_type=jnp.float32)
    enc = jnp.floor(pre + bs_ref[...])
    E = enc.astype(jnp.bfloat16).reshape(_C, _K2)
    t12 = jnp.dot(E, A12_ref[...],
                  preferred_element_type=jnp.float32)
    t1 = t12[:, :_PATCH_DIM]
    t2s = jnp.roll(t12[:, _PATCH_DIM:], -1, axis=0)
    tok = jnp.floor(t1 + t2s + bp_ref[...])
    m = jnp.mean(tok, axis=1, keepdims=True)
    c = tok - m
    v = jnp.mean(c * c, axis=1, keepdims=True)
    o_ref[0] = c * jax.lax.rsqrt(v + 1e-5) * lnw_ref[...] + lnb_ref[...]


@functools.partial(jax.jit, static_argnames=("interpret",))
def kernel(x, A_sample, b_sample, A_patch, b_patch, ln_weight, ln_bias,
           interpret=False):
    Ap = A_patch.reshape(_SAMPLE_DIM, _WINDOW, _PATCH_DIM)
    Ap = Ap.transpose(1, 0, 2)
    A1 = Ap[:_STEP].reshape(_K2, _PATCH_DIM)
    A2 = Ap[_STEP:].reshape(_K2, _PATCH_DIM)
    A12 = jnp.concatenate([A1, A2], axis=1).astype(jnp.bfloat16)
    xh = x.astype(jnp.bfloat16)
    Ash = A_sample.astype(jnp.bfloat16)
    bs = b_sample.reshape(1, _SAMPLE_DIM)
    bp = b_patch.reshape(1, _PATCH_DIM)
    lnw = ln_weight.reshape(1, _PATCH_DIM)
    lnb = ln_bias.reshape(1, _PATCH_DIM)

    out = pl.pallas_call(
        _tok_kernel,
        grid=(_B,),
        in_specs=[
            pl.BlockSpec((1, _V, _S), lambda b: (b, 0, 0)),
            pl.BlockSpec((_V, _SAMPLE_DIM), lambda b: (0, 0)),
            pl.BlockSpec((1, _SAMPLE_DIM), lambda b: (0, 0)),
            pl.BlockSpec((_K2, 2 * _PATCH_DIM), lambda b: (0, 0)),
            pl.BlockSpec((1, _PATCH_DIM), lambda b: (0, 0)),
            pl.BlockSpec((1, _PATCH_DIM), lambda b: (0, 0)),
            pl.BlockSpec((1, _PATCH_DIM), lambda b: (0, 0)),
        ],
        out_specs=pl.BlockSpec((1, _C, _PATCH_DIM), lambda b: (b, 0, 0)),
        out_shape=jax.ShapeDtypeStruct((_B, _C, _PATCH_DIM), jnp.float32),
        interpret=interpret,
    )(xh, Ash, bs, A12, bp, lnw, lnb)
    return out[:, :_T, :]


# speedup vs baseline: 4.8823x; 1.3973x over previous
import functools

import jax
import jax.numpy as jnp
from jax.experimental import pallas as pl

_B, _V, _S = 16, 64, 2048
_SAMPLE_DIM, _PATCH_DIM = 128, 256
_WINDOW, _STEP = 32, 16
_C = _S // _STEP
_T = (_S - _WINDOW + _STEP - 1) // _STEP
_K2 = _STEP * _SAMPLE_DIM
_BB = 4


def _tok_kernel(x_ref, As_ref, bs_ref, A12_ref, bp_ref, lnw_ref,
                lnb_ref, o_ref):
    As = As_ref[...]
    pres = [
        jax.lax.dot_general(x_ref[i], As, (((0,), (0,)), ((), ())),
                            preferred_element_type=jnp.float32)
        for i in range(_BB)
    ]
    pre = jnp.concatenate(pres, axis=0)
    enc = jnp.floor(pre + bs_ref[...])
    E = enc.astype(jnp.bfloat16).reshape(_BB * _C, _K2)
    t12 = jnp.dot(E, A12_ref[...],
                  preferred_element_type=jnp.float32)
    t2s = jnp.roll(t12[:, _PATCH_DIM:], -1, axis=0)
    tok = jnp.floor(t12[:, :_PATCH_DIM] + t2s + bp_ref[...])
    m = jnp.mean(tok, axis=1, keepdims=True)
    c = tok - m
    v = jnp.mean(c * c, axis=1, keepdims=True)
    o = c * jax.lax.rsqrt(v + 1e-5) * lnw_ref[...] + lnb_ref[...]
    for i in range(_BB):
        o_ref[i] = o[i * _C:i * _C + _T]


@functools.partial(jax.jit, static_argnames=("interpret",))
def kernel(x, A_sample, b_sample, A_patch, b_patch, ln_weight, ln_bias,
           interpret=False):
    Ap = A_patch.reshape(_SAMPLE_DIM, _WINDOW, _PATCH_DIM)
    Ap = Ap.transpose(1, 0, 2)
    A1 = Ap[:_STEP].reshape(_K2, _PATCH_DIM)
    A2 = Ap[_STEP:].reshape(_K2, _PATCH_DIM)
    A12 = jnp.concatenate([A1, A2], axis=1).astype(jnp.bfloat16)
    bs = b_sample.reshape(1, _SAMPLE_DIM)
    bp = b_patch.reshape(1, _PATCH_DIM)
    lnw = ln_weight.reshape(1, _PATCH_DIM)
    lnb = ln_bias.reshape(1, _PATCH_DIM)

    return pl.pallas_call(
        _tok_kernel,
        grid=(_B // _BB,),
        in_specs=[
            pl.BlockSpec((_BB, _V, _S), lambda b: (b, 0, 0)),
            pl.BlockSpec((_V, _SAMPLE_DIM), lambda b: (0, 0)),
            pl.BlockSpec((1, _SAMPLE_DIM), lambda b: (0, 0)),
            pl.BlockSpec((_K2, 2 * _PATCH_DIM), lambda b: (0, 0)),
            pl.BlockSpec((1, _PATCH_DIM), lambda b: (0, 0)),
            pl.BlockSpec((1, _PATCH_DIM), lambda b: (0, 0)),
            pl.BlockSpec((1, _PATCH_DIM), lambda b: (0, 0)),
        ],
        out_specs=pl.BlockSpec((_BB, _T, _PATCH_DIM), lambda b: (b, 0, 0)),
        out_shape=jax.ShapeDtypeStruct((_B, _T, _PATCH_DIM), jnp.float32),
        interpret=interpret,
    )(x, A_sample, bs, A12, bp, lnw, lnb)


# speedup vs baseline: 5.4967x; 1.1259x over previous
import functools

import jax
import jax.numpy as jnp
from jax.experimental import pallas as pl
from jax.experimental.pallas import tpu as pltpu

_B, _V, _S = 16, 64, 2048
_SAMPLE_DIM, _PATCH_DIM = 128, 256
_WINDOW, _STEP = 32, 16
_C = _S // _STEP
_T = (_S - _WINDOW + _STEP - 1) // _STEP
_K2 = _STEP * _SAMPLE_DIM
_BB = 4


def _tok_kernel(x_ref, As_ref, bs_ref, Ap_ref, bp_ref, lnw_ref,
                lnb_ref, o_ref, A12_ref):
    @pl.when(pl.program_id(0) == 0)
    def _build_a12():
        ap = Ap_ref[...].reshape(_SAMPLE_DIM, _WINDOW, _PATCH_DIM)
        A1 = jnp.concatenate([ap[:, w, :] for w in range(_STEP)], axis=0)
        A2 = jnp.concatenate([ap[:, w, :] for w in range(_STEP, _WINDOW)],
                             axis=0)
        A12_ref[...] = jnp.concatenate([A1, A2], axis=1).astype(jnp.bfloat16)

    As = As_ref[...]
    pres = [
        jax.lax.dot_general(x_ref[i], As, (((0,), (0,)), ((), ())),
                            preferred_element_type=jnp.float32)
        for i in range(_BB)
    ]
    pre = jnp.concatenate(pres, axis=0)
    enc = jnp.floor(pre + bs_ref[...])
    E = enc.astype(jnp.bfloat16).reshape(_BB * _C, _K2)
    t12 = jnp.dot(E, A12_ref[...],
                  preferred_element_type=jnp.float32)
    t2s = jnp.roll(t12[:, _PATCH_DIM:], -1, axis=0)
    tok = jnp.floor(t12[:, :_PATCH_DIM] + t2s + bp_ref[...])
    m = jnp.mean(tok, axis=1, keepdims=True)
    c = tok - m
    v = jnp.mean(c * c, axis=1, keepdims=True)
    o = c * jax.lax.rsqrt(v + 1e-5) * lnw_ref[...] + lnb_ref[...]
    for i in range(_BB):
        o_ref[i] = o[i * _C:i * _C + _T]


@functools.partial(jax.jit, static_argnames=("interpret",))
def kernel(x, A_sample, b_sample, A_patch, b_patch, ln_weight, ln_bias,
           interpret=False):
    bs = b_sample.reshape(1, _SAMPLE_DIM)
    bp = b_patch.reshape(1, _PATCH_DIM)
    lnw = ln_weight.reshape(1, _PATCH_DIM)
    lnb = ln_bias.reshape(1, _PATCH_DIM)

    return pl.pallas_call(
        _tok_kernel,
        grid=(_B // _BB,),
        in_specs=[
            pl.BlockSpec((_BB, _V, _S), lambda b: (b, 0, 0)),
            pl.BlockSpec((_V, _SAMPLE_DIM), lambda b: (0, 0)),
            pl.BlockSpec((1, _SAMPLE_DIM), lambda b: (0, 0)),
            pl.BlockSpec((_WINDOW * _SAMPLE_DIM, _PATCH_DIM),
                         lambda b: (0, 0)),
            pl.BlockSpec((1, _PATCH_DIM), lambda b: (0, 0)),
            pl.BlockSpec((1, _PATCH_DIM), lambda b: (0, 0)),
            pl.BlockSpec((1, _PATCH_DIM), lambda b: (0, 0)),
        ],
        out_specs=pl.BlockSpec((_BB, _T, _PATCH_DIM), lambda b: (b, 0, 0)),
        out_shape=jax.ShapeDtypeStruct((_B, _T, _PATCH_DIM), jnp.float32),
        scratch_shapes=[pltpu.VMEM((_K2, 2 * _PATCH_DIM), jnp.bfloat16)],
        interpret=interpret,
    )(x, A_sample, bs, A_patch, bp, lnw, lnb)
